# Initial kernel scaffold; baseline (speedup 1.0000x reference)
import jax
import jax.numpy as jnp
from jax.experimental import pallas as pl


def kernel(boxes0, boxes1, boxes2, boxes3, boxes4, boxes5, gt_boxes, pairs, default_boxes):
    raise NotImplementedError("write your pallas kernel here")


# speedup vs baseline: 395.6002x; 395.6002x over previous
import jax
import jax.numpy as jnp
from jax import lax
from jax.experimental import pallas as pl
from jax.experimental.pallas import tpu as pltpu
from jax.experimental.pallas import tpu_sc as plsc

_NDB = 8432
_B, _P, _G = 64, 8192, 256
_LVL_N = (5476, 2166, 600, 150, 36, 4)
_LVL_OFF = (0, 5476, 7642, 8242, 8392, 8428)
_NC, _NS, _L = 2, 16, 16
_NW = _NC * _NS
_BPW = _B // _NW


def _prep_body(db_ref, gt_ref, dbt_ref, gtt_ref):
    db = db_ref[...]
    dbt_ref[0:2, :] = db[0:2, :]
    dbt_ref[2:4, :] = jnp.log(db[2:4, :])
    dbt_ref[4:5, :] = 1.0 / db[3:4, :]
    dbt_ref[5:6, :] = 1.0 / db[2:3, :]
    gt = gt_ref[...]
    gtt_ref[:, 0:2, :] = gt[:, 0:2, :]
    gtt_ref[:, 2:4, :] = jnp.log(gt[:, 2:4, :])


def _sc_body(l0, l1, l2, l3, l4, l5, gtt_hbm, dbt_hbm, di_hbm, gi_hbm,
             out_hbm, flat_v, dbt_v, gtt_v, di_v, gi_v, out_v, sem):
    wid = lax.axis_index("c") * _NS + lax.axis_index("s")
    pltpu.sync_copy(dbt_hbm, dbt_v)
    cols = [jnp.full((_L,), c, jnp.int32) for c in range(6)]
    levels = (l0, l1, l2, l3, l4, l5)

    def batch_loss(b):
        copies = [pltpu.async_copy(lvl.at[b], flat_v.at[pl.ds(off * 4, n * 4)],
                                   sem)
                  for lvl, n, off in zip(levels, _LVL_N, _LVL_OFF)]
        copies.append(pltpu.async_copy(gtt_hbm.at[b], gtt_v, sem))
        copies.append(pltpu.async_copy(di_hbm.at[b], di_v, sem))
        copies.append(pltpu.async_copy(gi_hbm.at[b], gi_v, sem))
        for c in copies:
            c.wait()

        def step(i, acc):
            di = di_v[pl.ds(i * _L, _L)]
            gi = gi_v[pl.ds(i * _L, _L)]
            t0 = (plsc.load_gather(gtt_v, [cols[0], gi])
                  - plsc.load_gather(dbt_v, [cols[0], di])
                  ) * plsc.load_gather(dbt_v, [cols[4], di])
            t1 = (plsc.load_gather(gtt_v, [cols[1], gi])
                  - plsc.load_gather(dbt_v, [cols[1], di])
                  ) * plsc.load_gather(dbt_v, [cols[5], di])
            t2 = (plsc.load_gather(gtt_v, [cols[2], gi])
                  - plsc.load_gather(dbt_v, [cols[2], di]))
            t3 = (plsc.load_gather(gtt_v, [cols[3], gi])
                  - plsc.load_gather(dbt_v, [cols[3], di]))
            di4 = di * 4
            for c, t in enumerate((t0, t1, t2, t3)):
                p = plsc.load_gather(flat_v, [di4 + cols[c]])
                e = jnp.abs(p - t)
                acc = acc + jnp.where(e < 1.0, (0.5 * e) * e, e - 0.5)
            return acc

        return lax.fori_loop(0, _P // _L, step, jnp.zeros((_L,), jnp.float32))

    acc = batch_loss(wid * _BPW)
    for k in range(1, _BPW):
        acc = acc + batch_loss(wid * _BPW + k)
    out_v[...] = acc
    pltpu.sync_copy(out_v, out_hbm.at[wid])


def kernel(boxes0, boxes1, boxes2, boxes3, boxes4, boxes5, gt_boxes, pairs,
           default_boxes):
    levels = [b.reshape(_B, -1)
              for b in (boxes0, boxes1, boxes2, boxes3, boxes4, boxes5)]
    di = pairs[:, :, 0]
    gi = pairs[:, :, 1]

    dbt, gtt = pl.pallas_call(
        _prep_body,
        out_shape=[jax.ShapeDtypeStruct((6, _NDB), jnp.float32),
                   jax.ShapeDtypeStruct((_B, 4, _G), jnp.float32)],
    )(default_boxes.T, gt_boxes.transpose(0, 2, 1))

    sc = pl.kernel(
        _sc_body,
        out_type=jax.ShapeDtypeStruct((_NW, _L), jnp.float32),
        mesh=plsc.VectorSubcoreMesh(core_axis_name="c", subcore_axis_name="s"),
        compiler_params=pltpu.CompilerParams(needs_layout_passes=False,
                                             use_tc_tiling_on_sc=False),
        scratch_types=[
            pltpu.VMEM((_NDB * 4,), jnp.float32),
            pltpu.VMEM((6, _NDB), jnp.float32),
            pltpu.VMEM((4, _G), jnp.float32),
            pltpu.VMEM((_P,), jnp.int32),
            pltpu.VMEM((_P,), jnp.int32),
            pltpu.VMEM((_L,), jnp.float32),
            pltpu.SemaphoreType.DMA,
        ],
    )
    part = sc(*levels, gtt, dbt, di, gi)
    return jnp.sum(part) / jnp.float32(_B * _P * 4)
